# Initial kernel scaffold; baseline (speedup 1.0000x reference)
import jax
import jax.numpy as jnp
from jax.experimental import pallas as pl


def kernel(tokens, pos_emb):
    raise NotImplementedError("write your pallas kernel here")


# speedup vs baseline: 1.6774x; 1.6774x over previous
import jax
import jax.numpy as jnp
from jax.experimental import pallas as pl


def _add_block(t_ref, p_ref, o_ref):
    o_ref[...] = t_ref[...] + p_ref[...]


def kernel(tokens, pos_emb):
    B, L, D = tokens.shape
    BL = 512
    grid = (L // BL, B)
    return pl.pallas_call(
        _add_block,
        grid=grid,
        in_specs=[
            pl.BlockSpec((1, BL, D), lambda s, b: (b, s, 0)),
            pl.BlockSpec((BL, D), lambda s, b: (s, 0)),
        ],
        out_specs=pl.BlockSpec((1, BL, D), lambda s, b: (b, s, 0)),
        out_shape=jax.ShapeDtypeStruct((B, L, D), tokens.dtype),
    )(tokens, pos_emb)
